# Initial kernel scaffold; baseline (speedup 1.0000x reference)
import jax
import jax.numpy as jnp
from jax.experimental import pallas as pl


def kernel(x, W):
    raise NotImplementedError("write your pallas kernel here")


# speedup vs baseline: 13.3536x; 13.3536x over previous
import functools

import jax
import jax.numpy as jnp
from jax import lax
from jax.experimental import pallas as pl
from jax.experimental.pallas import tpu as pltpu
from jax.experimental.pallas import tpu_sc as plsc

N_FIELDS = 22
VOCAB = 12
EMB_DIM = 64
BATCH = 16384

TOTAL_ROWS = BATCH * N_FIELDS
NUM_WORKERS = 32
ROWS_PER_WORKER = TOTAL_ROWS // NUM_WORKERS
CHUNK = 128
NCHUNKS = ROWS_PER_WORKER // CHUNK
LANES = 16


def _body(xflat_hbm, table_hbm, out_hbm, xbuf, idxbuf, rowsbuf, sem):
    wid = lax.axis_index("s") * 2 + lax.axis_index("c")
    wbase = wid * ROWS_PER_WORKER
    lane = lax.iota(jnp.int32, 16)

    def chunk_body(g, carry):
        base = wbase + g * CHUNK
        pltpu.sync_copy(xflat_hbm.at[pl.ds(base, CHUNK)], xbuf)

        def vec_body(i, c2):
            s = i * LANES
            pos = (g * CHUNK + s) + lane
            off = (pos % N_FIELDS) * VOCAB
            idxbuf[pl.ds(s, LANES)] = xbuf[pl.ds(s, LANES)] + off
            return c2

        lax.fori_loop(0, CHUNK // LANES, vec_body, 0)
        pltpu.async_copy(table_hbm.at[idxbuf], rowsbuf, sem).wait()
        pltpu.sync_copy(rowsbuf, out_hbm.at[pl.ds(base, CHUNK)])
        return carry

    lax.fori_loop(0, NCHUNKS, chunk_body, 0)


@jax.jit
def _gather(xflat, table):
    mesh = plsc.VectorSubcoreMesh(core_axis_name="c", subcore_axis_name="s")
    return pl.kernel(
        _body,
        out_type=jax.ShapeDtypeStruct((TOTAL_ROWS, EMB_DIM), jnp.float32),
        mesh=mesh,
        scratch_types=[
            pltpu.VMEM((CHUNK,), jnp.int32),
            pltpu.VMEM((CHUNK,), jnp.int32),
            pltpu.VMEM((CHUNK, EMB_DIM), jnp.float32),
            pltpu.SemaphoreType.DMA,
        ],
        compiler_params=pltpu.CompilerParams(use_tc_tiling_on_sc=False),
    )(xflat, table)


def kernel(x, W):
    xflat = x.reshape(-1).astype(jnp.int32)
    table = W.reshape(N_FIELDS * VOCAB, EMB_DIM)
    out = _gather(xflat, table)
    return out.reshape(BATCH, N_FIELDS * EMB_DIM)


# speedup vs baseline: 13.4467x; 1.0070x over previous
import jax
import jax.numpy as jnp
from jax import lax
from jax.experimental import pallas as pl
from jax.experimental.pallas import tpu as pltpu
from jax.experimental.pallas import tpu_sc as plsc

N_FIELDS = 22
VOCAB = 12
EMB_DIM = 64
BATCH = 16384

TOTAL_ROWS = BATCH * N_FIELDS
NUM_WORKERS = 32
ROWS_PER_WORKER = TOTAL_ROWS // NUM_WORKERS
CHUNK = 128
NCHUNKS = ROWS_PER_WORKER // CHUNK
LANES = 16
NBUF = 8
NITER = NCHUNKS // NBUF


def _body(xflat_hbm, table_hbm, out_hbm, idxbuf, rowsbuf, *sems):
    gsem = sems[:NBUF]
    wsem = sems[NBUF:]
    wid = lax.axis_index("s") * 2 + lax.axis_index("c")
    wbase = wid * ROWS_PER_WORKER
    lane = lax.iota(jnp.int32, LANES)

    pltpu.sync_copy(xflat_hbm.at[pl.ds(wbase, ROWS_PER_WORKER)], idxbuf)

    def id_body(i, c):
        s = i * LANES
        off = ((s + lane) % N_FIELDS) * VOCAB
        idxbuf[pl.ds(s, LANES)] = idxbuf[pl.ds(s, LANES)] + off
        return c

    lax.fori_loop(0, ROWS_PER_WORKER // LANES, id_body, 0)

    def g_start(slot, g):
        pltpu.async_copy(
            table_hbm.at[idxbuf.at[pl.ds(g * CHUNK, CHUNK)]],
            rowsbuf.at[slot], gsem[slot])

    def g_wait(slot, g):
        pltpu.make_async_copy(
            table_hbm.at[idxbuf.at[pl.ds(g * CHUNK, CHUNK)]],
            rowsbuf.at[slot], gsem[slot]).wait()

    def w_start(slot, g):
        pltpu.async_copy(
            rowsbuf.at[slot],
            out_hbm.at[pl.ds(wbase + g * CHUNK, CHUNK)], wsem[slot])

    def w_wait(slot, g):
        pltpu.make_async_copy(
            rowsbuf.at[slot],
            out_hbm.at[pl.ds(wbase + g * CHUNK, CHUNK)], wsem[slot]).wait()

    for b in range(NBUF):
        g_start(b, b)

    def block(k, c):
        for b in range(NBUF):
            g_wait(b, k * NBUF + b)
            w_start(b, k * NBUF + b)

        @pl.when(k < NITER - 1)
        def _():
            for b in range(NBUF):
                w_wait(b, k * NBUF + b)
                g_start(b, (k + 1) * NBUF + b)

        return c

    lax.fori_loop(0, NITER, block, 0)

    for b in range(NBUF):
        w_wait(b, (NITER - 1) * NBUF + b)


@jax.jit
def _gather(xflat, table):
    mesh = plsc.VectorSubcoreMesh(core_axis_name="c", subcore_axis_name="s")
    return pl.kernel(
        _body,
        out_type=jax.ShapeDtypeStruct((TOTAL_ROWS, EMB_DIM), jnp.float32),
        mesh=mesh,
        scratch_types=[
            pltpu.VMEM((ROWS_PER_WORKER,), jnp.int32),
            pltpu.VMEM((NBUF, CHUNK, EMB_DIM), jnp.float32),
        ] + [pltpu.SemaphoreType.DMA] * (2 * NBUF),
        compiler_params=pltpu.CompilerParams(use_tc_tiling_on_sc=False),
    )(xflat, table)


def kernel(x, W):
    xflat = x.reshape(-1).astype(jnp.int32)
    table = W.reshape(N_FIELDS * VOCAB, EMB_DIM)
    out = _gather(xflat, table)
    return out.reshape(BATCH, N_FIELDS * EMB_DIM)


# speedup vs baseline: 13.6036x; 1.0117x over previous
import jax
import jax.numpy as jnp
from jax import lax
from jax.experimental import pallas as pl
from jax.experimental.pallas import tpu as pltpu
from jax.experimental.pallas import tpu_sc as plsc

N_FIELDS = 22
VOCAB = 12
EMB_DIM = 64
BATCH = 16384

TOTAL_ROWS = BATCH * N_FIELDS
NUM_WORKERS = 32
ROWS_PER_WORKER = TOTAL_ROWS // NUM_WORKERS
CHUNK = 128
NCHUNKS = ROWS_PER_WORKER // CHUNK
LANES = 16
NBUF = 8
NITER = NCHUNKS // NBUF


def _body(xflat_hbm, table_hbm, out_hbm, idxbuf, rowsbuf, *sems):
    gsem = sems[:NBUF]
    wsem = sems[NBUF:]
    wid = lax.axis_index("s") * 2 + lax.axis_index("c")
    wbase = wid * ROWS_PER_WORKER
    lane = lax.iota(jnp.int32, LANES)

    pltpu.sync_copy(xflat_hbm.at[pl.ds(wbase, ROWS_PER_WORKER)], idxbuf)

    offs = [((j * LANES + lane) % N_FIELDS) * VOCAB for j in range(11)]

    def id_body(r, c):
        base = r * (11 * LANES)
        for j in range(11):
            s = base + j * LANES
            idxbuf[pl.ds(s, LANES)] = idxbuf[pl.ds(s, LANES)] + offs[j]
        return c

    lax.fori_loop(0, ROWS_PER_WORKER // (11 * LANES), id_body, 0)

    def g_start(slot, g):
        pltpu.async_copy(
            table_hbm.at[idxbuf.at[pl.ds(g * CHUNK, CHUNK)]],
            rowsbuf.at[slot], gsem[slot])

    def g_wait(slot, g):
        pltpu.make_async_copy(
            table_hbm.at[idxbuf.at[pl.ds(g * CHUNK, CHUNK)]],
            rowsbuf.at[slot], gsem[slot]).wait()

    def w_start(slot, g):
        pltpu.async_copy(
            rowsbuf.at[slot],
            out_hbm.at[pl.ds(wbase + g * CHUNK, CHUNK)], wsem[slot])

    def w_wait(slot, g):
        pltpu.make_async_copy(
            rowsbuf.at[slot],
            out_hbm.at[pl.ds(wbase + g * CHUNK, CHUNK)], wsem[slot]).wait()

    for b in range(NBUF):
        g_start(b, b)

    def block(k, c):
        for b in range(NBUF):
            g_wait(b, k * NBUF + b)
            w_start(b, k * NBUF + b)

        @pl.when(k < NITER - 1)
        def _():
            for b in range(NBUF):
                w_wait(b, k * NBUF + b)
                g_start(b, (k + 1) * NBUF + b)

        return c

    lax.fori_loop(0, NITER, block, 0)

    for b in range(NBUF):
        w_wait(b, (NITER - 1) * NBUF + b)


@jax.jit
def _gather(xflat, table):
    mesh = plsc.VectorSubcoreMesh(core_axis_name="c", subcore_axis_name="s")
    return pl.kernel(
        _body,
        out_type=jax.ShapeDtypeStruct((TOTAL_ROWS, EMB_DIM), jnp.float32),
        mesh=mesh,
        scratch_types=[
            pltpu.VMEM((ROWS_PER_WORKER,), jnp.int32),
            pltpu.VMEM((NBUF, CHUNK, EMB_DIM), jnp.float32),
        ] + [pltpu.SemaphoreType.DMA] * (2 * NBUF),
        compiler_params=pltpu.CompilerParams(use_tc_tiling_on_sc=False),
    )(xflat, table)


def kernel(x, W):
    xflat = x.reshape(-1).astype(jnp.int32)
    table = W.reshape(N_FIELDS * VOCAB, EMB_DIM)
    out = _gather(xflat, table)
    return out.reshape(BATCH, N_FIELDS * EMB_DIM)
